# baseline (speedup 1.0000x reference)
import functools

import jax
import jax.numpy as jnp
from jax import lax
from jax.experimental import pallas as pl
from jax.experimental.pallas import tpu as pltpu
from jax.experimental.pallas import tpu_sc as plsc

N = 10000
E = 320000
D = 128

NC = 2
NS = 16
NW = NC * NS
CHUNK = 125
NCHUNK = 80
KB = 16
NB = NCHUNK // KB
NBUF = 2
EPT = NCHUNK * CHUNK
EPAD = NW * EPT
NA = N
ROWS_A = 632
ROWS_LAST = NA - (NS - 1) * ROWS_A

_mesh = plsc.VectorSubcoreMesh(core_axis_name="c", subcore_axis_name="s")

_AGG_SCRATCH = (
    (pltpu.VMEM((KB, CHUNK), jnp.int32),
     pltpu.VMEM((KB, CHUNK), jnp.int32))
    + tuple(pltpu.VMEM((CHUNK, D), jnp.float32)
            for _ in range(NBUF))
    + (pltpu.VMEM_SHARED((NA, D), jnp.float32),)
    + tuple(pltpu.SemaphoreType.DMA for _ in range(NBUF))
)


def _striped(s, copy_fn):
  row0 = pl.multiple_of(s * ROWS_A, 8)

  @pl.when(s < NS - 1)
  def _():
    copy_fn(row0, ROWS_A)

  @pl.when(s == NS - 1)
  def _():
    copy_fn(row0, ROWS_LAST)


def _agg_pipeline(h_hbm, src_hbm, dst_hbm, wid, idx_s, idx_d, rows, sems,
                  acc_sh):
  for blk in range(NB):
    pltpu.sync_copy(src_hbm.at[wid, blk], idx_s)
    pltpu.sync_copy(dst_hbm.at[wid, blk], idx_d)

    for b in range(NBUF):
      pltpu.async_copy(h_hbm.at[idx_s.at[b]], rows[b], sems[b])

    def body(j, carry):
      for b in range(NBUF):
        ch = NBUF * j + b
        pltpu.make_async_copy(h_hbm.at[idx_s.at[ch]], rows[b], sems[b]).wait()
        pltpu.sync_copy(rows[b], acc_sh.at[idx_d.at[ch]], add=True)

        @pl.when(ch + NBUF < KB)
        def _():
          pltpu.async_copy(h_hbm.at[idx_s.at[ch + NBUF]], rows[b], sems[b])
      return carry

    lax.fori_loop(0, KB // NBUF, body, 0)
    for ch in range((KB // NBUF) * NBUF, KB):
      pltpu.make_async_copy(h_hbm.at[idx_s.at[ch]], rows[ch % NBUF],
                            sems[ch % NBUF]).wait()
      pltpu.sync_copy(rows[ch % NBUF], acc_sh.at[idx_d.at[ch]], add=True)


@functools.partial(
    pl.kernel,
    out_type=(jax.ShapeDtypeStruct((NC, NA, D), jnp.float32),
              jax.ShapeDtypeStruct((NC, NA, D), jnp.float32)),
    mesh=_mesh,
    scratch_types=_AGG_SCRATCH,
)
def _sc_agg_deg(h_hbm, src_hbm, dst_hbm, z_rows, ones_hbm, sums_out, deg_out,
                idx_s, idx_d, *rest):
  rows = rest[:NBUF]
  acc_sh = rest[NBUF]
  sems = rest[NBUF + 1:]
  c = lax.axis_index("c")
  s = lax.axis_index("s")
  wid = s * NC + c

  _striped(s, lambda r, n: pltpu.sync_copy(z_rows.at[pl.ds(r, n)],
                                           acc_sh.at[pl.ds(r, n)]))
  plsc.subcore_barrier()

  _agg_pipeline(h_hbm, src_hbm, dst_hbm, wid, idx_s, idx_d, rows, sems,
                acc_sh)
  plsc.subcore_barrier()

  def _flush(r, n):
    pltpu.sync_copy(acc_sh.at[pl.ds(r, n)], sums_out.at[c, pl.ds(r, n)])
    pltpu.sync_copy(z_rows.at[pl.ds(r, n)], acc_sh.at[pl.ds(r, n)])
  _striped(s, _flush)
  ones_v = rows[-1]
  pltpu.sync_copy(ones_hbm, ones_v)
  plsc.subcore_barrier()

  for blk in range(NB):
    pltpu.sync_copy(dst_hbm.at[wid, blk], idx_d)

    def dbody(j, carry):
      pltpu.sync_copy(ones_v, acc_sh.at[idx_d.at[j]], add=True)
      return carry

    lax.fori_loop(0, KB, dbody, 0)

  plsc.subcore_barrier()
  _striped(s, lambda r, n: pltpu.sync_copy(acc_sh.at[pl.ds(r, n)],
                                           deg_out.at[c, pl.ds(r, n)]))


@functools.partial(
    pl.kernel,
    out_type=jax.ShapeDtypeStruct((NC, NA, D), jnp.float32),
    mesh=_mesh,
    scratch_types=_AGG_SCRATCH,
)
def _sc_agg(h_hbm, src_hbm, dst_hbm, z_rows, sums_out, idx_s, idx_d, *rest):
  rows = rest[:NBUF]
  acc_sh = rest[NBUF]
  sems = rest[NBUF + 1:]
  c = lax.axis_index("c")
  s = lax.axis_index("s")
  wid = s * NC + c

  _striped(s, lambda r, n: pltpu.sync_copy(z_rows.at[pl.ds(r, n)],
                                           acc_sh.at[pl.ds(r, n)]))
  plsc.subcore_barrier()

  _agg_pipeline(h_hbm, src_hbm, dst_hbm, wid, idx_s, idx_d, rows, sems,
                acc_sh)
  plsc.subcore_barrier()

  _striped(s, lambda r, n: pltpu.sync_copy(acc_sh.at[pl.ds(r, n)],
                                           sums_out.at[c, pl.ds(r, n)]))


RB = 2000


def _dense_body(x_ref, sp_ref, dp_ref, w12_ref, w2t_ref, b12_ref,
                gamma_ref, beta_ref, out_ref, *, with_ln):
  x = x_ref[...]
  ssum = sp_ref[0] + sp_ref[1]
  deg = dp_ref[0, :, 0:1] + dp_ref[1, :, 0:1]
  mean = ssum / jnp.maximum(deg, 1.0)
  out = (jnp.dot(x, w12_ref[...], preferred_element_type=jnp.float32)
         + b12_ref[...]
         - jnp.dot(mean, w2t_ref[...], preferred_element_type=jnp.float32))
  out = jnp.where(deg > 0.0, out, x)
  if with_ln:
    h = jnp.maximum(out, 0.0)
    mu = jnp.mean(h, axis=1, keepdims=True)
    var = jnp.mean((h - mu) * (h - mu), axis=1, keepdims=True)
    out = (h - mu) * lax.rsqrt(var + 1e-5) * gamma_ref[...] + beta_ref[...]
  out_ref[...] = out


def _make_dense(with_ln):
  body = functools.partial(_dense_body, with_ln=with_ln)
  return pl.pallas_call(
      body,
      grid=(N // RB,),
      in_specs=[
          pl.BlockSpec((RB, D), lambda i: (i, 0)),
          pl.BlockSpec((NC, RB, D), lambda i: (0, i, 0)),
          pl.BlockSpec((NC, RB, D), lambda i: (0, i, 0)),
          pl.BlockSpec((D, D), lambda i: (0, 0)),
          pl.BlockSpec((D, D), lambda i: (0, 0)),
          pl.BlockSpec((1, D), lambda i: (0, 0)),
          pl.BlockSpec((1, D), lambda i: (0, 0)),
          pl.BlockSpec((1, D), lambda i: (0, 0)),
      ],
      out_specs=pl.BlockSpec((RB, D), lambda i: (i, 0)),
      out_shape=jax.ShapeDtypeStruct((N, D), jnp.float32),
  )


_dense_ln = _make_dense(True)
_dense_out = _make_dense(False)


def kernel(x, edge_index, W1_0, b1_0, W2_0, b2_0, gamma, beta,
           W1_1, b1_1, W2_1, b2_1):
  src = edge_index[0].astype(jnp.int32)
  dst = edge_index[1].astype(jnp.int32)
  src = jnp.concatenate([src, jnp.zeros((EPAD - E,), jnp.int32)])
  dst = jnp.concatenate([dst, jnp.full((EPAD - E,), N, jnp.int32)])
  src4 = src.reshape(NW, NB, KB, CHUNK)
  dst4 = dst.reshape(NW, NB, KB, CHUNK)

  z_rows = jnp.zeros((NA, D), jnp.float32)
  ones = jnp.ones((CHUNK, D), jnp.float32)

  w12_0 = (W1_0 + W2_0).T
  w2t_0 = W2_0.T
  b12_0 = (b1_0 + b2_0).reshape(1, D)
  w12_1 = (W1_1 + W2_1).T
  w2t_1 = W2_1.T
  b12_1 = (b1_1 + b2_1).reshape(1, D)
  gamma2 = gamma.reshape(1, D)
  beta2 = beta.reshape(1, D)

  sums0, degp = _sc_agg_deg(x, src4, dst4, z_rows, ones)
  h1 = _dense_ln(x, sums0, degp, w12_0, w2t_0, b12_0, gamma2, beta2)
  sums1 = _sc_agg(h1, src4, dst4, z_rows)
  out = _dense_out(h1, sums1, degp, w12_1, w2t_1, b12_1, gamma2, beta2)
  return out


# speedup vs baseline: 1.1327x; 1.1327x over previous
import functools

import jax
import jax.numpy as jnp
from jax import lax
from jax.experimental import pallas as pl
from jax.experimental.pallas import tpu as pltpu
from jax.experimental.pallas import tpu_sc as plsc

N = 10000
E = 320000
D = 128

NC = 2
NS = 16
NW = NC * NS
CHUNK = 80
NCHUNK = 125
KB = 5
NB = NCHUNK // KB
NBUF = 3
EPT = NCHUNK * CHUNK
EPAD = NW * EPT
NA = N
ROWS_A = 632
ROWS_LAST = NA - (NS - 1) * ROWS_A

_mesh = plsc.VectorSubcoreMesh(core_axis_name="c", subcore_axis_name="s")

_AGG_SCRATCH = (
    tuple(pltpu.VMEM((KB, CHUNK), jnp.int32)
          for _ in range(2))
    + tuple(pltpu.VMEM((KB, CHUNK), jnp.int32)
            for _ in range(2))
    + tuple(pltpu.VMEM((CHUNK, D), jnp.float32)
            for _ in range(NBUF))
    + (pltpu.VMEM_SHARED((NA, D), jnp.float32),)
    + tuple(pltpu.SemaphoreType.DMA for _ in range(NBUF + 4))
)


def _striped(s, copy_fn):
  row0 = pl.multiple_of(s * ROWS_A, 8)

  @pl.when(s < NS - 1)
  def _():
    copy_fn(row0, ROWS_A)

  @pl.when(s == NS - 1)
  def _():
    copy_fn(row0, ROWS_LAST)


def _agg_pipeline(h_hbm, src_hbm, dst_hbm, wid, idx_s, idx_d, rows, sems,
                  sem_is, sem_id, acc_sh):
  def issue_idx(b):
    pltpu.async_copy(src_hbm.at[wid, b], idx_s[b % 2], sem_is[b % 2])
    pltpu.async_copy(dst_hbm.at[wid, b], idx_d[b % 2], sem_id[b % 2])

  def wait_idx(b):
    pltpu.make_async_copy(src_hbm.at[wid, b], idx_s[b % 2],
                          sem_is[b % 2]).wait()
    pltpu.make_async_copy(dst_hbm.at[wid, b], idx_d[b % 2],
                          sem_id[b % 2]).wait()

  def gather(ch, b):
    pltpu.async_copy(h_hbm.at[idx_s[(ch // KB) % 2].at[ch % KB]], rows[b],
                     sems[b])

  pltpu.sync_copy(src_hbm.at[wid, 0], idx_s[0])
  pltpu.sync_copy(dst_hbm.at[wid, 0], idx_d[0])
  if NB > 1:
    issue_idx(1)
  for b in range(NBUF):
    gather(b, b)

  for ch in range(NCHUNK):
    blk = ch // KB
    b = ch % NBUF
    pltpu.make_async_copy(h_hbm.at[idx_s[blk % 2].at[ch % KB]], rows[b],
                          sems[b]).wait()
    pltpu.sync_copy(rows[b], acc_sh.at[idx_d[blk % 2].at[ch % KB]], add=True)
    if ch % KB == KB - 1 and blk + 2 < NB:
      issue_idx(blk + 2)
    g = ch + NBUF
    if g < NCHUNK:
      if g % KB == 0:
        wait_idx(g // KB)
      gather(g, g % NBUF)


def _unpack(rest):
  idx_s = rest[0:2]
  idx_d = rest[2:4]
  rows = rest[4:4 + NBUF]
  acc_sh = rest[4 + NBUF]
  sems = rest[5 + NBUF:5 + 2 * NBUF]
  sem_is = rest[5 + 2 * NBUF:7 + 2 * NBUF]
  sem_id = rest[7 + 2 * NBUF:9 + 2 * NBUF]
  return idx_s, idx_d, rows, acc_sh, sems, sem_is, sem_id


def _sc_agg_deg(h_hbm, src_hbm, dst_hbm, z_rows, ones_hbm, sums_out, deg_out,
                *rest):
  idx_s, idx_d, rows, acc_sh, sems, sem_is, sem_id = _unpack(rest)
  c = lax.axis_index("c")
  s = lax.axis_index("s")
  wid = s * NC + c

  _striped(s, lambda r, n: pltpu.sync_copy(z_rows.at[pl.ds(r, n)],
                                           acc_sh.at[pl.ds(r, n)]))
  plsc.subcore_barrier()

  _agg_pipeline(h_hbm, src_hbm, dst_hbm, wid, idx_s, idx_d, rows, sems,
                sem_is, sem_id, acc_sh)
  plsc.subcore_barrier()

  def _flush(r, n):
    pltpu.sync_copy(acc_sh.at[pl.ds(r, n)], sums_out.at[c, pl.ds(r, n)])
    pltpu.sync_copy(z_rows.at[pl.ds(r, n)], acc_sh.at[pl.ds(r, n)])
  _striped(s, _flush)
  ones_v = rows[-1]
  pltpu.sync_copy(ones_hbm, ones_v)
  plsc.subcore_barrier()

  pltpu.sync_copy(dst_hbm.at[wid, 0], idx_d[0])
  if NB > 1:
    pltpu.async_copy(dst_hbm.at[wid, 1], idx_d[1], sem_id[1])
  for blk in range(NB):
    if blk >= 1:
      pltpu.make_async_copy(dst_hbm.at[wid, blk], idx_d[blk % 2],
                            sem_id[blk % 2]).wait()
    for j in range(KB):
      pltpu.sync_copy(ones_v, acc_sh.at[idx_d[blk % 2].at[j]], add=True)
    if blk + 2 < NB:
      pltpu.async_copy(dst_hbm.at[wid, blk + 2], idx_d[blk % 2],
                       sem_id[blk % 2])

  plsc.subcore_barrier()
  _striped(s, lambda r, n: pltpu.sync_copy(acc_sh.at[pl.ds(r, n)],
                                           deg_out.at[c, pl.ds(r, n)]))


_sc_agg_deg = functools.partial(
    pl.kernel,
    out_type=(jax.ShapeDtypeStruct((NC, NA, D), jnp.float32),
              jax.ShapeDtypeStruct((NC, NA, D), jnp.float32)),
    mesh=_mesh,
    scratch_types=_AGG_SCRATCH,
)(_sc_agg_deg)


@functools.partial(
    pl.kernel,
    out_type=jax.ShapeDtypeStruct((NC, NA, D), jnp.float32),
    mesh=_mesh,
    scratch_types=_AGG_SCRATCH,
)
def _sc_agg(h_hbm, src_hbm, dst_hbm, z_rows, sums_out, *rest):
  idx_s, idx_d, rows, acc_sh, sems, sem_is, sem_id = _unpack(rest)
  c = lax.axis_index("c")
  s = lax.axis_index("s")
  wid = s * NC + c

  _striped(s, lambda r, n: pltpu.sync_copy(z_rows.at[pl.ds(r, n)],
                                           acc_sh.at[pl.ds(r, n)]))
  plsc.subcore_barrier()

  _agg_pipeline(h_hbm, src_hbm, dst_hbm, wid, idx_s, idx_d, rows, sems,
                sem_is, sem_id, acc_sh)
  plsc.subcore_barrier()

  _striped(s, lambda r, n: pltpu.sync_copy(acc_sh.at[pl.ds(r, n)],
                                           sums_out.at[c, pl.ds(r, n)]))


RB = 2000


def _dense_body(x_ref, sp_ref, dp_ref, w12_ref, w2t_ref, b12_ref,
                gamma_ref, beta_ref, out_ref, *, with_ln):
  x = x_ref[...]
  ssum = sp_ref[0] + sp_ref[1]
  deg = dp_ref[0, :, 0:1] + dp_ref[1, :, 0:1]
  mean = ssum / jnp.maximum(deg, 1.0)
  out = (jnp.dot(x, w12_ref[...], preferred_element_type=jnp.float32)
         + b12_ref[...]
         - jnp.dot(mean, w2t_ref[...], preferred_element_type=jnp.float32))
  out = jnp.where(deg > 0.0, out, x)
  if with_ln:
    h = jnp.maximum(out, 0.0)
    mu = jnp.mean(h, axis=1, keepdims=True)
    var = jnp.mean((h - mu) * (h - mu), axis=1, keepdims=True)
    out = (h - mu) * lax.rsqrt(var + 1e-5) * gamma_ref[...] + beta_ref[...]
  out_ref[...] = out


def _make_dense(with_ln):
  body = functools.partial(_dense_body, with_ln=with_ln)
  return pl.pallas_call(
      body,
      grid=(N // RB,),
      in_specs=[
          pl.BlockSpec((RB, D), lambda i: (i, 0)),
          pl.BlockSpec((NC, RB, D), lambda i: (0, i, 0)),
          pl.BlockSpec((NC, RB, D), lambda i: (0, i, 0)),
          pl.BlockSpec((D, D), lambda i: (0, 0)),
          pl.BlockSpec((D, D), lambda i: (0, 0)),
          pl.BlockSpec((1, D), lambda i: (0, 0)),
          pl.BlockSpec((1, D), lambda i: (0, 0)),
          pl.BlockSpec((1, D), lambda i: (0, 0)),
      ],
      out_specs=pl.BlockSpec((RB, D), lambda i: (i, 0)),
      out_shape=jax.ShapeDtypeStruct((N, D), jnp.float32),
  )


_dense_ln = _make_dense(True)
_dense_out = _make_dense(False)


def kernel(x, edge_index, W1_0, b1_0, W2_0, b2_0, gamma, beta,
           W1_1, b1_1, W2_1, b2_1):
  src = edge_index[0].astype(jnp.int32)
  dst = edge_index[1].astype(jnp.int32)
  src = jnp.concatenate([src, jnp.zeros((EPAD - E,), jnp.int32)])
  dst = jnp.concatenate([dst, jnp.full((EPAD - E,), N, jnp.int32)])
  src4 = src.reshape(NW, NB, KB, CHUNK)
  dst4 = dst.reshape(NW, NB, KB, CHUNK)

  z_rows = jnp.zeros((NA, D), jnp.float32)
  ones = jnp.ones((CHUNK, D), jnp.float32)

  w12_0 = (W1_0 + W2_0).T
  w2t_0 = W2_0.T
  b12_0 = (b1_0 + b2_0).reshape(1, D)
  w12_1 = (W1_1 + W2_1).T
  w2t_1 = W2_1.T
  b12_1 = (b1_1 + b2_1).reshape(1, D)
  gamma2 = gamma.reshape(1, D)
  beta2 = beta.reshape(1, D)

  sums0, degp = _sc_agg_deg(x, src4, dst4, z_rows, ones)
  h1 = _dense_ln(x, sums0, degp, w12_0, w2t_0, b12_0, gamma2, beta2)
  sums1 = _sc_agg(h1, src4, dst4, z_rows)
  out = _dense_out(h1, sums1, degp, w12_1, w2t_1, b12_1, gamma2, beta2)
  return out
